# baseline (speedup 1.0000x reference)
import functools

import jax
import jax.numpy as jnp
from jax import lax
from jax.experimental import pallas as pl
from jax.experimental.pallas import tpu as pltpu
from jax.experimental.pallas import tpu_sc as plsc

PAD = 1
BSZ = 4
SEQ = 4096
D = 1024
G = BSZ * SEQ
NW = 32
JW = SEQ // NW
R = 32
NCH = JW // R
L = 16
NTOK = BSZ * JW


def _make_kernel():
    mesh = plsc.VectorSubcoreMesh(core_axis_name="c", subcore_axis_name="s")

    @functools.partial(
        pl.kernel,
        mesh=mesh,
        out_type=jax.ShapeDtypeStruct((G * D,), jnp.float32),
        scratch_types=[
            pltpu.VMEM((NTOK,), jnp.int32),
            pltpu.VMEM((R * D,), jnp.float32),
            pltpu.VMEM((R * D,), jnp.float32),
            pltpu.VMEM((D,), jnp.float32),
            pltpu.SemaphoreType.DMA,
            pltpu.SemaphoreType.DMA,
            pltpu.SemaphoreType.DMA,
            pltpu.SemaphoreType.DMA,
            pltpu.SemaphoreType.DMA,
        ],
    )
    def k(inp_hbm, w_hbm, out_hbm, tok_v, buf0, buf1, zrow,
          sem_r0, sem_r1, sem_w0, sem_w1, sem_t):
        wid = lax.axis_index("s") * 2 + lax.axis_index("c")
        jlo = wid * JW

        tcps = [
            pltpu.async_copy(inp_hbm.at[pl.ds(b * SEQ + jlo, JW)],
                             tok_v.at[pl.ds(b * JW, JW)], sem_t)
            for b in range(BSZ)
        ]
        zv = jnp.zeros((L,), jnp.float32)

        def zbody(i, carry):
            zrow[pl.ds(i * L, L)] = zv
            return carry

        lax.fori_loop(0, D // L, zbody, jnp.int32(0))

        bufs = (buf0, buf1)
        rsems = (sem_r0, sem_r1)
        wsems = (sem_w0, sem_w1)

        def read(c):
            return pltpu.async_copy(
                w_hbm.at[pl.ds((jlo + c * R + PAD + 1) * D, R * D)],
                bufs[c % 2], rsems[c % 2])

        rcps = {0: read(0), 1: read(1)}
        wcps = {}
        for c in range(NCH):
            rcps[c].wait()
            wcps[c] = [
                pltpu.async_copy(
                    bufs[c % 2],
                    out_hbm.at[pl.ds((b * SEQ + jlo + c * R) * D, R * D)],
                    wsems[c % 2])
                for b in range(BSZ)
            ]
            if c + 2 < NCH:
                for cp in wcps[c]:
                    cp.wait()
                rcps[c + 2] = read(c + 2)

        for cp in tcps:
            cp.wait()

        def scan_body(g, acc):
            tok = tok_v[pl.ds(g * L, L)]
            return acc | jnp.where(tok == PAD, 1, 0)

        padv = lax.fori_loop(0, NTOK // L, scan_body,
                             jnp.zeros((L,), jnp.int32))
        anypad = padv[0]
        for l in range(1, L):
            anypad = anypad | padv[l]

        for c in range(NCH - 2, NCH):
            for cp in wcps[c]:
                cp.wait()

        @pl.when(anypad > 0)
        def _fixup():
            def body(g, carry):
                tok = tok_v[pl.ds(g * L, L)]
                for l in range(L):
                    @pl.when(tok[l] == PAD)
                    def _():
                        t = g * L + l
                        b = t // JW
                        r = t - b * JW
                        pltpu.sync_copy(
                            zrow,
                            out_hbm.at[pl.ds((b * SEQ + jlo + r) * D, D)])
                return carry

            lax.fori_loop(0, NTOK // L, body, jnp.int32(0))

    return k


_embed = _make_kernel()


@jax.jit
def kernel(input, weights):
    out = _embed(input.reshape(-1), weights.reshape(-1))
    return out.reshape(BSZ, SEQ, D)


# speedup vs baseline: 2.8553x; 2.8553x over previous
import functools

import jax
import jax.numpy as jnp
from jax import lax
from jax.experimental import pallas as pl
from jax.experimental.pallas import tpu as pltpu
from jax.experimental.pallas import tpu_sc as plsc

PAD = 1
BSZ = 4
SEQ = 4096
D = 1024
G = BSZ * SEQ
NW = 32
JW = SEQ // NW
R = 32
NCH = JW // R
L = 16
NTOK = BSZ * JW


def _make_kernel():
    mesh = plsc.VectorSubcoreMesh(core_axis_name="c", subcore_axis_name="s")

    @functools.partial(
        pl.kernel,
        mesh=mesh,
        out_type=jax.ShapeDtypeStruct((G, D), jnp.float32),
        scratch_types=[
            pltpu.VMEM((NTOK,), jnp.int32),
            pltpu.VMEM((R,), jnp.int32),
            pltpu.VMEM((R,), jnp.int32),
            pltpu.VMEM((R, D), jnp.float32),
            pltpu.VMEM((R, D), jnp.float32),
            pltpu.SemaphoreType.DMA,
            pltpu.SemaphoreType.DMA,
            pltpu.SemaphoreType.DMA,
            pltpu.SemaphoreType.DMA,
            pltpu.SemaphoreType.DMA,
        ],
    )
    def k(inp_hbm, w_hbm, out_hbm, tok_v, idx0, idx1, buf0, buf1,
          sem_r0, sem_r1, sem_w0, sem_w1, sem_t):
        wid = lax.axis_index("s") * 2 + lax.axis_index("c")
        jlo = wid * JW

        for b in range(BSZ):
            pltpu.async_copy(inp_hbm.at[pl.ds(b * SEQ + jlo, JW)],
                             tok_v.at[pl.ds(b * JW, JW)], sem_t).wait()

        def scan_body(g, acc):
            tok = tok_v[pl.ds(g * L, L)]
            return acc | jnp.where(tok == PAD, 1, 0)

        padv = lax.fori_loop(0, NTOK // L, scan_body,
                             jnp.zeros((L,), jnp.int32))
        anypad = padv[0]
        for l in range(1, L):
            anypad = anypad | padv[l]

        idxs = (idx0, idx1)
        bufs = (buf0, buf1)
        rsems = (sem_r0, sem_r1)
        wsems = (sem_w0, sem_w1)
        iota = lax.iota(jnp.int32, L)

        @pl.when(anypad == 0)
        def _fast():
            def gather(c):
                p = c % 2
                for v in range(R // L):
                    idxs[p][pl.ds(v * L, L)] = (
                        jlo + c * R + v * L + (PAD + 1) + iota)
                return pltpu.async_copy(w_hbm.at[idxs[p]], bufs[p], rsems[p])

            rcps = {0: gather(0), 1: gather(1)}
            wcps = {}
            for c in range(NCH):
                rcps[c].wait()
                wcps[c] = [
                    pltpu.async_copy(
                        bufs[c % 2],
                        out_hbm.at[pl.ds(b * SEQ + jlo + c * R, R)],
                        wsems[c % 2])
                    for b in range(BSZ)
                ]
                if c + 2 < NCH:
                    for cp in wcps[c]:
                        cp.wait()
                    rcps[c + 2] = gather(c + 2)
            for c in range(NCH - 2, NCH):
                for cp in wcps[c]:
                    cp.wait()

        @pl.when(anypad != 0)
        def _slow():
            steps = [(c, b) for c in range(NCH) for b in range(BSZ)]

            def gather(s):
                c, b = steps[s]
                p = s % 2
                for v in range(R // L):
                    tok = tok_v[pl.ds(b * JW + c * R + v * L, L)]
                    pos = jlo + c * R + v * L + (PAD + 1) + iota
                    idxs[p][pl.ds(v * L, L)] = (
                        jnp.where(tok != PAD, pos, PAD))
                return pltpu.async_copy(w_hbm.at[idxs[p]], bufs[p], rsems[p])

            rcps = {0: gather(0), 1: gather(1)}
            wcps = {}
            for s in range(len(steps)):
                c, b = steps[s]
                rcps[s].wait()
                wcps[s] = pltpu.async_copy(
                    bufs[s % 2],
                    out_hbm.at[pl.ds(b * SEQ + jlo + c * R, R)],
                    wsems[s % 2])
                if s + 2 < len(steps):
                    wcps[s].wait()
                    rcps[s + 2] = gather(s + 2)
            wcps[len(steps) - 2].wait()
            wcps[len(steps) - 1].wait()

    return k


_embed = _make_kernel()


@jax.jit
def kernel(input, weights):
    out = _embed(input.reshape(-1), weights)
    return out.reshape(BSZ, SEQ, D)


# speedup vs baseline: 3.0719x; 1.0759x over previous
import functools

import jax
import jax.numpy as jnp
from jax import lax
from jax.experimental import pallas as pl
from jax.experimental.pallas import tpu as pltpu
from jax.experimental.pallas import tpu_sc as plsc

PAD = 1
BSZ = 4
SEQ = 4096
D = 1024
G = BSZ * SEQ
NW = 32
JW = SEQ // NW
R = 32
NCH = JW // R
L = 16


def _make_kernel():
    mesh = plsc.VectorSubcoreMesh(core_axis_name="c", subcore_axis_name="s")

    @functools.partial(
        pl.kernel,
        mesh=mesh,
        out_type=jax.ShapeDtypeStruct((G, D), jnp.float32),
        scratch_types=[
            pltpu.VMEM((BSZ, JW), jnp.int32),
            pltpu.VMEM((R,), jnp.int32),
            pltpu.VMEM((R,), jnp.int32),
            pltpu.VMEM((R, D), jnp.float32),
            pltpu.VMEM((R, D), jnp.float32),
            pltpu.SemaphoreType.DMA,
            pltpu.SemaphoreType.DMA,
            pltpu.SemaphoreType.DMA,
            pltpu.SemaphoreType.DMA,
            pltpu.SemaphoreType.DMA,
        ],
    )
    def k(inp_hbm, w_hbm, out_hbm, tok_v, idx0, idx1, buf0, buf1,
          sem_r0, sem_r1, sem_w0, sem_w1, sem_t):
        wid = lax.axis_index("s") * 2 + lax.axis_index("c")
        jlo = wid * JW

        idxs = (idx0, idx1)
        bufs = (buf0, buf1)
        rsems = (sem_r0, sem_r1)
        wsems = (sem_w0, sem_w1)
        iota = lax.iota(jnp.int32, L)

        def gather_affine(c):
            p = c % 2
            for v in range(R // L):
                idxs[p][pl.ds(v * L, L)] = (
                    jlo + c * R + v * L + (PAD + 1) + iota)
            return pltpu.async_copy(w_hbm.at[idxs[p]], bufs[p], rsems[p])

        rcps = {0: gather_affine(0), 1: gather_affine(1)}
        tcp = pltpu.async_copy(inp_hbm.at[:, pl.ds(jlo, JW)], tok_v, sem_t)

        tcp.wait()
        padv = jnp.zeros((L,), jnp.int32)
        for b in range(BSZ):
            def scan_body(g, acc):
                tok = tok_v[b, pl.ds(g * L, L)]
                return acc | jnp.where(tok == PAD, 1, 0)
            padv = lax.fori_loop(0, JW // L, scan_body, padv)
        anypad = padv[0]
        for l in range(1, L):
            anypad = anypad | padv[l]

        @pl.when(anypad == 0)
        def _fast():
            wcps = {}
            for c in range(NCH):
                rcps[c].wait()
                wcps[c] = [
                    pltpu.async_copy(
                        bufs[c % 2],
                        out_hbm.at[pl.ds(b * SEQ + jlo + c * R, R)],
                        wsems[c % 2])
                    for b in range(BSZ)
                ]
                if c + 2 < NCH:
                    for cp in wcps[c]:
                        cp.wait()
                    rcps[c + 2] = gather_affine(c + 2)
            for c in range(NCH - 2, NCH):
                for cp in wcps[c]:
                    cp.wait()

        @pl.when(anypad != 0)
        def _slow():
            rcps[0].wait()
            rcps[1].wait()
            for b in range(BSZ):
                def chunk_body(c, carry):
                    def fill(v, carry2):
                        tok = tok_v[b, pl.ds(c * R + v * L, L)]
                        pos = jlo + c * R + v * L + (PAD + 1) + iota
                        idx0[pl.ds(v * L, L)] = jnp.where(tok != PAD, pos, PAD)
                        return carry2

                    lax.fori_loop(0, R // L, fill, jnp.int32(0))
                    pltpu.async_copy(w_hbm.at[idx0], buf0, sem_r0).wait()
                    dst = pl.multiple_of(b * SEQ + jlo + c * R, 8)
                    pltpu.async_copy(buf0, out_hbm.at[pl.ds(dst, R)],
                                     sem_w0).wait()
                    return carry

                lax.fori_loop(0, NCH, chunk_body, jnp.int32(0))

    return k


_embed = _make_kernel()


@jax.jit
def kernel(input, weights):
    out = _embed(input, weights)
    return out.reshape(BSZ, SEQ, D)


# speedup vs baseline: 3.0827x; 1.0035x over previous
import functools

import jax
import jax.numpy as jnp
from jax import lax
from jax.experimental import pallas as pl
from jax.experimental.pallas import tpu as pltpu
from jax.experimental.pallas import tpu_sc as plsc

PAD = 1
BSZ = 4
SEQ = 4096
D = 1024
G = BSZ * SEQ
NW = 32
JW = SEQ // NW
R = 32
NCH = JW // R
L = 16


def _make_kernel():
    mesh = plsc.VectorSubcoreMesh(core_axis_name="c", subcore_axis_name="s")

    @functools.partial(
        pl.kernel,
        mesh=mesh,
        out_type=jax.ShapeDtypeStruct((G, D), jnp.float32),
        scratch_types=[
            pltpu.VMEM((BSZ, JW), jnp.int32),
            pltpu.VMEM((R,), jnp.int32),
            pltpu.VMEM((R,), jnp.int32),
            pltpu.VMEM((R, D), jnp.float32),
            pltpu.VMEM((R, D), jnp.float32),
            pltpu.SemaphoreType.DMA,
            pltpu.SemaphoreType.DMA,
            pltpu.SemaphoreType.DMA,
            pltpu.SemaphoreType.DMA,
            pltpu.SemaphoreType.DMA,
        ],
    )
    def k(inp_hbm, w_hbm, out_hbm, tok_v, idx0, idx1, buf0, buf1,
          sem_r0, sem_r1, sem_w0, sem_w1, sem_t):
        wid = lax.axis_index("s") * 2 + lax.axis_index("c")
        jlo = wid * JW

        idxs = (idx0, idx1)
        bufs = (buf0, buf1)
        rsems = (sem_r0, sem_r1)
        wsems = (sem_w0, sem_w1)
        iota = lax.iota(jnp.int32, L)

        def gather_affine(c):
            p = c % 2
            for v in range(R // L):
                idxs[p][pl.ds(v * L, L)] = (
                    jlo + c * R + v * L + (PAD + 1) + iota)
            return pltpu.async_copy(w_hbm.at[idxs[p]], bufs[p], rsems[p])

        rcps = {0: gather_affine(0), 1: gather_affine(1)}
        tcp = pltpu.async_copy(inp_hbm.at[:, pl.ds(jlo, JW)], tok_v, sem_t)

        tcp.wait()
        GPB = JW // L

        def scan_body(g, acc):
            b = g // GPB
            tok = tok_v[b, pl.ds((g - b * GPB) * L, L)]
            return acc | jnp.where(tok == PAD, 1, 0)

        padv = lax.fori_loop(0, BSZ * GPB, scan_body,
                             jnp.zeros((L,), jnp.int32))
        anypad = padv[0]
        for l in range(1, L):
            anypad = anypad | padv[l]

        @pl.when(anypad == 0)
        def _fast():
            wcps = {}
            for c in range(NCH):
                rcps[c].wait()
                wcps[c] = [
                    pltpu.async_copy(
                        bufs[c % 2],
                        out_hbm.at[pl.ds(b * SEQ + jlo + c * R, R)],
                        wsems[c % 2])
                    for b in range(BSZ)
                ]
                if c + 2 < NCH:
                    for cp in wcps[c]:
                        cp.wait()
                    rcps[c + 2] = gather_affine(c + 2)
            for c in range(NCH - 2, NCH):
                for cp in wcps[c]:
                    cp.wait()

        @pl.when(anypad != 0)
        def _slow():
            rcps[0].wait()
            rcps[1].wait()

            def step(s, carry):
                b = s // NCH
                c = s - b * NCH

                def fill(v, carry2):
                    tok = tok_v[b, pl.ds(c * R + v * L, L)]
                    pos = jlo + c * R + v * L + (PAD + 1) + iota
                    idx0[pl.ds(v * L, L)] = jnp.where(tok != PAD, pos, PAD)
                    return carry2

                lax.fori_loop(0, R // L, fill, jnp.int32(0))
                pltpu.async_copy(w_hbm.at[idx0], buf0, sem_r0).wait()
                dst = pl.multiple_of(b * SEQ + jlo + c * R, 8)
                pltpu.async_copy(buf0, out_hbm.at[pl.ds(dst, R)],
                                 sem_w0).wait()
                return carry

            lax.fori_loop(0, BSZ * NCH, step, jnp.int32(0))

    return k


_embed = _make_kernel()


@jax.jit
def kernel(input, weights):
    out = _embed(input, weights)
    return out.reshape(BSZ, SEQ, D)
